# Initial kernel scaffold; baseline (speedup 1.0000x reference)
import jax
import jax.numpy as jnp
from jax.experimental import pallas as pl


def kernel(queries, candidates):
    raise NotImplementedError("write your pallas kernel here")


# speedup vs baseline: 1.9994x; 1.9994x over previous
import functools

import jax
import jax.numpy as jnp
from jax.experimental import pallas as pl
from jax.experimental.pallas import tpu as pltpu

_K = 10
_NEG_INF = float("-inf")
_INT_MAX = jnp.iinfo(jnp.int32).max


def _extract_topk(work_s, work_i, k):
    outs, outi = [], []
    for _ in range(k):
        m = jnp.max(work_s, axis=1, keepdims=True)
        sel = jnp.min(
            jnp.where(work_s == m, work_i, _INT_MAX), axis=1, keepdims=True
        )
        outs.append(m)
        outi.append(sel)
        work_s = jnp.where(work_i == sel, _NEG_INF, work_s)
    return jnp.concatenate(outs, axis=1), jnp.concatenate(outi, axis=1)


def _stream_body(c_real, chunk, nsteps, q_ref, c_ref, s_out, i_out,
                 bs_ref, bi_ref):
    step = pl.program_id(0)

    @pl.when(step == 0)
    def _init():
        bs_ref[...] = jnp.full(bs_ref.shape, _NEG_INF, jnp.float32)
        bi_ref[...] = jnp.full(bi_ref.shape, _INT_MAX, jnp.int32)

    q = q_ref[...]
    c = c_ref[...]
    s = jax.lax.dot_general(
        q, c, (((1,), (1,)), ((), ())),
        preferred_element_type=jnp.float32,
    )
    gids = jax.lax.broadcasted_iota(jnp.int32, s.shape, 1) + step * chunk
    s = jnp.where(gids < c_real, s, _NEG_INF)

    work_s = jnp.concatenate([bs_ref[...], s], axis=1)
    work_i = jnp.concatenate([bi_ref[...], gids], axis=1)
    new_s, new_i = _extract_topk(work_s, work_i, _K)

    bs_ref[:, 0:_K] = new_s
    bi_ref[:, 0:_K] = new_i

    @pl.when(step == nsteps - 1)
    def _fin():
        s_out[...] = new_s
        i_out[...] = new_i


@jax.jit
def kernel(queries, candidates):
    qn, d = queries.shape
    cn = candidates.shape[0]
    chunk = 2048
    nsteps = -(-cn // chunk)
    cpad = nsteps * chunk
    if cpad != cn:
        candidates = jnp.pad(candidates, ((0, cpad - cn), (0, 0)))

    body = functools.partial(_stream_body, cn, chunk, nsteps)
    out_s, out_i = pl.pallas_call(
        body,
        grid=(nsteps,),
        in_specs=[
            pl.BlockSpec((qn, d), lambda i: (0, 0)),
            pl.BlockSpec((chunk, d), lambda i: (i, 0)),
        ],
        out_specs=[
            pl.BlockSpec((qn, _K), lambda i: (0, 0)),
            pl.BlockSpec((qn, _K), lambda i: (0, 0)),
        ],
        out_shape=[
            jax.ShapeDtypeStruct((qn, _K), jnp.float32),
            jax.ShapeDtypeStruct((qn, _K), jnp.int32),
        ],
        scratch_shapes=[
            pltpu.VMEM((qn, 16), jnp.float32),
            pltpu.VMEM((qn, 16), jnp.int32),
        ],
    )(queries, candidates)
    return (out_s, out_i)


# speedup vs baseline: 10.2096x; 5.1062x over previous
import functools

import jax
import jax.numpy as jnp
from jax.experimental import pallas as pl
from jax.experimental.pallas import tpu as pltpu
from jax.experimental.pallas import tpu_sc as plsc

_K = 10
_SEL = 12
_B = 16
_GRP = 128
_CHUNK = 4096
_BPC = _CHUNK // _B
_CP = 1 << 20
_NCH = _CP // _CHUNK
_NB = _CP // _B
_NS = _NB // _GRP
_CPQ = _SEL * _B
_QG = 64
_M2CH = 8192

_NEG_INF = float("-inf")
_INT_MAX = jnp.iinfo(jnp.int32).max


def _extract(work_s, work_i, k):
    outs, outi = [], []
    for _ in range(k):
        m = jnp.max(work_s, axis=1, keepdims=True)
        sel = jnp.min(
            jnp.where(work_s == m, work_i, _INT_MAX), axis=1, keepdims=True
        )
        outs.append(m)
        outi.append(sel)
        work_s = jnp.where(work_i == sel, _NEG_INF, work_s)
    return jnp.concatenate(outs, axis=1), jnp.concatenate(outi, axis=1)


def _p1_body(c_real, q_ref, c_ref, m_ref, m2_ref):
    ci = pl.program_id(0)
    q = q_ref[...]
    jlane = jax.lax.broadcasted_iota(jnp.int32, (q.shape[0], _BPC), 1)
    m = None
    for t in range(_B):
        st = jax.lax.dot_general(
            q, c_ref[t], (((1,), (1,)), ((), ())),
            preferred_element_type=jnp.float32,
        )
        gid = ci * _CHUNK + jlane * _B + t
        st = jnp.where(gid < c_real, st, _NEG_INF)
        m = st if m is None else jnp.maximum(m, st)
    m_ref[...] = m
    m2_ref[...] = jnp.concatenate(
        [jnp.max(m[:, u * _GRP:(u + 1) * _GRP], axis=1, keepdims=True)
         for u in range(_BPC // _GRP)], axis=1)[None]


def _p2a_body(m2_ref, rows_ref, sup_ref):
    m2 = m2_ref[...]
    ids = jax.lax.broadcasted_iota(jnp.int32, m2.shape, 1)
    _, sup = _extract(m2, ids, _SEL)
    qi = jax.lax.broadcasted_iota(jnp.int32, sup.shape, 0)
    rows_ref[...] = qi * _NS + sup
    sup_ref[...] = sup


def _p2b_body(mg_ref, sup_ref, blk_ref, cid_ref):
    mg = mg_ref[...]
    sup = sup_ref[...]
    giota = jax.lax.broadcasted_iota(jnp.int32, (sup.shape[0], _GRP), 1)
    bids = jnp.concatenate(
        [sup[:, j:j + 1] * _GRP + giota for j in range(_SEL)], axis=1)
    _, blk = _extract(mg, bids, _SEL)
    blk_ref[...] = blk
    tiota = jax.lax.broadcasted_iota(jnp.int32, (sup.shape[0], _B), 1)
    cid_ref[...] = jnp.concatenate(
        [blk[:, j:j + 1] * _B + tiota for j in range(_SEL)], axis=1)


def _p4_body(c_real, q_ref, g_ref, cid_ref, s_out, i_out):
    q = q_ref[...]
    g = g_ref[...]
    s = jax.lax.dot_general(
        q, g, (((1,), (1,)), ((), ())),
        preferred_element_type=jnp.float32,
    )
    lane = jax.lax.broadcasted_iota(jnp.int32, s.shape, 1)
    row = jax.lax.broadcasted_iota(jnp.int32, s.shape, 0)
    ids = jnp.broadcast_to(cid_ref[...].reshape(1, _QG * _CPQ), s.shape)
    own = (lane >= row * _CPQ) & (lane < row * _CPQ + _CPQ)
    valid = own & (ids < c_real)
    ws = jnp.where(valid, s, _NEG_INF)
    wi = jnp.where(valid, ids, _INT_MAX)
    outs, outi = _extract(ws, wi, _K)
    s_out[...] = outs
    i_out[...] = outi


def _sc_gather(data, idx_flat, width, window):
    n = idx_flat.shape[1]
    mesh = plsc.VectorSubcoreMesh(core_axis_name="c", subcore_axis_name="s")

    @pl.kernel(
        out_type=jax.ShapeDtypeStruct((n, width), data.dtype), mesh=mesh)
    def k(x_hbm, i_hbm, o_hbm):
        def body(i_vmem, o_vmem):
            pltpu.sync_copy(x_hbm.at[i_vmem.at[0]], o_vmem)

        pltpu.emit_pipeline(
            body,
            grid=(n // window,),
            in_specs=[pl.BlockSpec((1, window), lambda i: (0, i))],
            out_specs=[pl.BlockSpec((window, width), lambda i: (i, 0))],
            core_axis_name=("c", "s"),
            dimension_semantics=(pltpu.PARALLEL,),
        )(i_hbm, o_hbm)

    return k(data, idx_flat)


@jax.jit
def kernel(queries, candidates):
    qn, d = queries.shape
    cn = candidates.shape[0]
    cpad = jnp.pad(candidates, ((0, _CP - cn), (0, 0)))
    cmem = jnp.transpose(cpad.reshape(_NB, _B, d), (1, 0, 2))

    spc = _BPC // _GRP
    m, m2_3d = pl.pallas_call(
        functools.partial(_p1_body, cn),
        grid=(_NCH,),
        in_specs=[
            pl.BlockSpec((qn, d), lambda i: (0, 0)),
            pl.BlockSpec((_B, _BPC, d), lambda i: (0, i, 0)),
        ],
        out_specs=[
            pl.BlockSpec((qn, _BPC), lambda i: (0, i)),
            pl.BlockSpec((1, qn, spc), lambda i: (i, 0, 0)),
        ],
        out_shape=[
            jax.ShapeDtypeStruct((qn, _NB), jnp.float32),
            jax.ShapeDtypeStruct((_NCH, qn, spc), jnp.float32),
        ],
    )(queries, cmem)
    m2 = jnp.transpose(m2_3d, (1, 0, 2)).reshape(qn, _NS)

    rows, sup = pl.pallas_call(
        _p2a_body,
        in_specs=[pl.BlockSpec((qn, _NS), lambda: (0, 0))],
        out_specs=[
            pl.BlockSpec((qn, _SEL), lambda: (0, 0)),
            pl.BlockSpec((qn, _SEL), lambda: (0, 0)),
        ],
        out_shape=[
            jax.ShapeDtypeStruct((qn, _SEL), jnp.int32),
            jax.ShapeDtypeStruct((qn, _SEL), jnp.int32),
        ],
    )(m2)

    mg = _sc_gather(m.reshape(qn * _NS, _GRP), rows.reshape(1, qn * _SEL),
                    _GRP, 128)

    blk, cids = pl.pallas_call(
        _p2b_body,
        in_specs=[
            pl.BlockSpec((qn, _SEL * _GRP), lambda: (0, 0)),
            pl.BlockSpec((qn, _SEL), lambda: (0, 0)),
        ],
        out_specs=[
            pl.BlockSpec((qn, _SEL), lambda: (0, 0)),
            pl.BlockSpec((qn, _CPQ), lambda: (0, 0)),
        ],
        out_shape=[
            jax.ShapeDtypeStruct((qn, _SEL), jnp.int32),
            jax.ShapeDtypeStruct((qn, _CPQ), jnp.int32),
        ],
    )(mg.reshape(qn, _SEL * _GRP), sup)

    g = _sc_gather(cpad.reshape(_NB, _B * d), blk.reshape(1, qn * _SEL),
                   _B * d, 128)

    ng = qn // _QG
    out_s, out_i = pl.pallas_call(
        functools.partial(_p4_body, cn),
        grid=(ng,),
        in_specs=[
            pl.BlockSpec((_QG, d), lambda i: (i, 0)),
            pl.BlockSpec((_QG * _CPQ, d), lambda i: (i, 0)),
            pl.BlockSpec((1, 1, _QG * _CPQ), lambda i: (i, 0, 0)),
        ],
        out_specs=[
            pl.BlockSpec((_QG, _K), lambda i: (i, 0)),
            pl.BlockSpec((_QG, _K), lambda i: (i, 0)),
        ],
        out_shape=[
            jax.ShapeDtypeStruct((qn, _K), jnp.float32),
            jax.ShapeDtypeStruct((qn, _K), jnp.int32),
        ],
    )(queries, g.reshape(qn * _CPQ, d), cids.reshape(ng, 1, _QG * _CPQ))
    return (out_s, out_i)
